# baseline (speedup 1.0000x reference)
import jax
import jax.numpy as jnp
from jax.experimental import pallas as pl
from jax.experimental.pallas import tpu as pltpu

_SEL_CHUNK = 512


def _sigmoid(x):
    return 0.5 * jnp.tanh(0.5 * x) + 0.5


def _swish(x):
    return x * _sigmoid(x)


def _bn_affine(gamma, beta, mean, var, eps):
    s = gamma / jnp.sqrt(var + eps)
    return s, beta - mean * s


def _bdot(a, b):
    return jnp.dot(a, b, preferred_element_type=jnp.float32)


def _make_fused_kernel(ns, mid, cout, cin, l_out, K, L):
    inv_l = 1.0 / float(l_out)
    lc = (cout - cin) // 2
    rc = cout - cin - lc
    R1 = ns * mid
    R3 = ns * cout
    nch = L // _SEL_CHUNK

    def body(x_ref, sel_ref, colA_ref, colB_ref, w1_ref, w2_ref,
             w3_ref, wf1_ref, wf2_ref, ones_ref, o_ref):
        w1 = w1_ref[...]
        w2 = w2_ref[...]
        w3 = w3_ref[...]
        wf1 = wf1_ref[...]
        wf2 = wf2_ref[...]
        ones = ones_ref[...]
        cA = jnp.tile(colA_ref[...], (1, ns, 1))
        s1, t1, s2, t2, s3, t3, bf1 = [cA[i] for i in range(7)]
        cB = jnp.tile(colB_ref[...], (1, ns, 1))
        b3, bf2 = cB[0], cB[1]
        x16 = x_ref[0].astype(jnp.bfloat16)
        xc = [x16[:, t * _SEL_CHUNK:(t + 1) * _SEL_CHUNK] for t in range(nch)]
        xe = jnp.concatenate([_bdot(c, sel_ref[0]) for c in xc], axis=1)
        xo = jnp.concatenate([_bdot(c, sel_ref[1]) for c in xc], axis=1)
        h = []
        for xs in (xe, xo):
            a = _swish(s1 * xs + t1).astype(jnp.bfloat16)
            h.append(_swish(s2 * _bdot(w1, a) + t2).astype(jnp.bfloat16))
        he, ho = h
        z1c = jnp.zeros((R1, 1), jnp.bfloat16)
        taps = (
            jnp.concatenate([z1c, ho[:, :l_out - 1]], axis=1),
            he,
            ho,
            jnp.concatenate([he[:, 1:], z1c], axis=1),
            jnp.concatenate([ho[:, 1:], z1c], axis=1),
        )
        y2 = _bdot(w2[:, 0:R1], taps[0])
        for k in range(1, K):
            y2 = y2 + _bdot(w2[:, k * R1:(k + 1) * R1], taps[k])
        h3 = _swish(s3 * y2 + t3).astype(jnp.bfloat16)
        y3 = _bdot(w3, h3) + b3
        se_b = _bdot(y3.astype(jnp.bfloat16), ones) * inv_l
        zz = _swish(_bdot(wf1, se_b.astype(jnp.bfloat16)) + bf1)
        z2 = _bdot(wf2, zz.astype(jnp.bfloat16)) + bf2
        gate = _sigmoid(z2)[:, 0:1]
        ident = jnp.maximum(xe, xo)
        idp = jnp.pad(ident.reshape(ns, cin, l_out),
                      ((0, 0), (lc, rc), (0, 0))).reshape(R3, l_out)
        o_ref[...] = (y3 * gate + idp).reshape(ns, cout, l_out)
    return body


def kernel(x, bn1_g, bn1_b, bn1_m, bn1_v, conv1_w, conv1_b,
           bn2_g, bn2_b, bn2_m, bn2_v, conv2_w, conv2_b,
           bn3_g, bn3_b, bn3_m, bn3_v, conv3_w, conv3_b,
           se_fc1_w, se_fc1_b, se_fc2_w, se_fc2_b):
    K, stride, groups = 5, 2, 2
    bn_eps = 1e-5
    N, Cin, L = x.shape
    mid = conv1_w.shape[0]
    Cout = conv3_w.shape[0]
    half = se_fc1_w.shape[0]
    cin_g = mid // groups

    s1, t1 = _bn_affine(bn1_g, bn1_b, bn1_m, bn1_v, bn_eps)
    s2, t2 = _bn_affine(bn2_g, bn2_b, bn2_m, bn2_v, bn_eps)
    s3, t3 = _bn_affine(bn3_g, bn3_b, bn3_m, bn3_v, bn_eps)
    t2 = t2 + s2 * conv1_b
    t3 = t3 + s3 * conv2_b

    L_out = -(-L // stride)
    p = max(0, (L_out - 1) * stride + K - L)
    assert p // 2 == 1 and L % 2 == 0 and stride == 2 and L % _SEL_CHUNK == 0

    ns = next(c for c in (8, 4, 2, 1) if N % c == 0)
    eye = jnp.eye(ns, dtype=jnp.float32)

    def kron(w):
        a, b = w.shape
        return (jnp.einsum('ij,ab->iajb', eye, w.astype(jnp.float32))
                .reshape(ns * a, ns * b).astype(jnp.bfloat16))

    w2f = conv2_w.astype(jnp.float32)
    w2full = jnp.zeros((mid, mid, K), jnp.float32)
    for g in range(groups):
        c0 = g * cin_g
        w2full = w2full.at[c0:c0 + cin_g, c0:c0 + cin_g, :].set(w2f[c0:c0 + cin_g])
    w2b = (jnp.einsum('ij,abk->iakjb', eye, w2full)
           .reshape(ns * mid, K * ns * mid).astype(jnp.bfloat16))

    colA = jnp.stack([s1, t1, s2, t2, s3, t3,
                      se_fc1_b.astype(jnp.float32)]).reshape(7, mid, 1)
    colB = jnp.stack([conv3_b.astype(jnp.float32),
                      se_fc2_b.astype(jnp.float32)]).reshape(2, Cout, 1)

    li = jnp.arange(_SEL_CHUNK)[:, None]
    qi = jnp.arange(_SEL_CHUNK // stride)[None, :]
    sel = jnp.stack([(li == stride * qi).astype(jnp.bfloat16),
                     (li == stride * qi + 1).astype(jnp.bfloat16)])

    xs = x.reshape(N // ns, ns * Cin, L)
    grid = (N // ns,)
    bs = pl.BlockSpec
    R1, R3 = ns * mid, ns * Cout
    ones_se = jnp.ones((L_out, 128), jnp.bfloat16)

    out = pl.pallas_call(
        _make_fused_kernel(ns, mid, Cout, Cin, L_out, K, L),
        out_shape=jax.ShapeDtypeStruct((N, Cout, L_out), jnp.float32),
        grid=grid,
        in_specs=[
            bs((1, ns * Cin, L), lambda n: (n, 0, 0)),
            bs((2, _SEL_CHUNK, _SEL_CHUNK // stride), lambda n: (0, 0, 0)),
            bs((7, mid, 1), lambda n: (0, 0, 0)),
            bs((2, Cout, 1), lambda n: (0, 0, 0)),
            bs((R1, ns * Cin), lambda n: (0, 0)),
            bs((R1, K * R1), lambda n: (0, 0)),
            bs((R3, R1), lambda n: (0, 0)),
            bs((ns * half, R3), lambda n: (0, 0)),
            bs((R3, ns * half), lambda n: (0, 0)),
            bs((L_out, 128), lambda n: (0, 0)),
        ],
        out_specs=bs((ns, Cout, L_out), lambda n: (n, 0, 0)),
        compiler_params=pltpu.CompilerParams(
            dimension_semantics=("parallel",)),
    )(xs, sel, colA, colB, kron(conv1_w[:, :, 0]), w2b,
      kron(conv3_w[:, :, 0]), kron(se_fc1_w), kron(se_fc2_w), ones_se)
    return out


# speedup vs baseline: 1.0594x; 1.0594x over previous
import jax
import jax.numpy as jnp
from jax.experimental import pallas as pl
from jax.experimental.pallas import tpu as pltpu

_SEL_CHUNK = 512


def _sigmoid(x):
    return 0.5 * jnp.tanh(0.5 * x) + 0.5


def _swish(x):
    return x * _sigmoid(x)


def _bn_affine(gamma, beta, mean, var, eps):
    s = gamma / jnp.sqrt(var + eps)
    return s, beta - mean * s


def _bdot(a, b):
    return jnp.dot(a, b, preferred_element_type=jnp.float32)


def _make_fused_kernel(ns, mid, cout, cin, l_out, K, L):
    inv_l = 1.0 / float(l_out)
    lc = (cout - cin) // 2
    rc = cout - cin - lc
    R1 = ns * mid
    R3 = ns * cout
    nch = L // _SEL_CHUNK

    def body(x_ref, sel_ref, colA_ref, colB_ref, w1_ref, w2_ref,
             w3_ref, wf1_ref, wf2_ref, ones_ref, o_ref):
        w1 = w1_ref[...]
        w2 = w2_ref[...]
        w3 = w3_ref[...]
        wf1 = wf1_ref[...]
        wf2 = wf2_ref[...]
        ones = ones_ref[...]
        cA = jnp.tile(colA_ref[...], (1, ns, 1))
        s1, t1, s2, t2, s3, t3, bf1 = [cA[i] for i in range(7)]
        cB = jnp.tile(colB_ref[...], (1, ns, 1))
        b3, bf2 = cB[0], cB[1]
        x16 = x_ref[0].astype(jnp.bfloat16)
        xc = [x16[:, t * _SEL_CHUNK:(t + 1) * _SEL_CHUNK] for t in range(nch)]
        xe = jnp.concatenate([_bdot(c, sel_ref[0]) for c in xc], axis=1)
        xo = jnp.concatenate([_bdot(c, sel_ref[1]) for c in xc], axis=1)
        h = []
        for xs in (xe, xo):
            a = _swish(s1 * xs + t1).astype(jnp.bfloat16)
            h.append(_swish(s2 * _bdot(w1, a) + t2).astype(jnp.bfloat16))
        he, ho = h
        z1c = jnp.zeros((R1, 1), jnp.bfloat16)
        taps = (
            jnp.concatenate([z1c, ho[:, :l_out - 1]], axis=1),
            he,
            ho,
            jnp.concatenate([he[:, 1:], z1c], axis=1),
            jnp.concatenate([ho[:, 1:], z1c], axis=1),
        )
        y2 = _bdot(w2[:, 0:R1], taps[0])
        for k in range(1, K):
            y2 = y2 + _bdot(w2[:, k * R1:(k + 1) * R1], taps[k])
        h3 = _swish(s3 * y2 + t3).astype(jnp.bfloat16)
        y3 = _bdot(w3, h3) + b3
        se_b = _bdot(y3.astype(jnp.bfloat16), ones) * inv_l
        zz = _swish(_bdot(wf1, se_b.astype(jnp.bfloat16)) + bf1)
        z2 = _bdot(wf2, zz.astype(jnp.bfloat16)) + bf2
        gate = _sigmoid(z2)[:, 0:1]
        ident = jnp.maximum(xe, xo)
        idp = jnp.pad(ident.reshape(ns, cin, l_out),
                      ((0, 0), (lc, rc), (0, 0))).reshape(R3, l_out)
        o_ref[...] = (y3 * gate + idp).reshape(ns, cout, l_out)
    return body


def kernel(x, bn1_g, bn1_b, bn1_m, bn1_v, conv1_w, conv1_b,
           bn2_g, bn2_b, bn2_m, bn2_v, conv2_w, conv2_b,
           bn3_g, bn3_b, bn3_m, bn3_v, conv3_w, conv3_b,
           se_fc1_w, se_fc1_b, se_fc2_w, se_fc2_b):
    K, stride, groups = 5, 2, 2
    bn_eps = 1e-5
    N, Cin, L = x.shape
    mid = conv1_w.shape[0]
    Cout = conv3_w.shape[0]
    half = se_fc1_w.shape[0]
    cin_g = mid // groups

    s1, t1 = _bn_affine(bn1_g, bn1_b, bn1_m, bn1_v, bn_eps)
    s2, t2 = _bn_affine(bn2_g, bn2_b, bn2_m, bn2_v, bn_eps)
    s3, t3 = _bn_affine(bn3_g, bn3_b, bn3_m, bn3_v, bn_eps)
    t2 = t2 + s2 * conv1_b
    t3 = t3 + s3 * conv2_b

    L_out = -(-L // stride)
    p = max(0, (L_out - 1) * stride + K - L)
    assert p // 2 == 1 and L % 2 == 0 and stride == 2 and L % _SEL_CHUNK == 0

    ns = next(c for c in (16, 8, 4, 2, 1) if N % c == 0)
    eye = jnp.eye(ns, dtype=jnp.float32)

    def kron(w):
        a, b = w.shape
        return (jnp.einsum('ij,ab->iajb', eye, w.astype(jnp.float32))
                .reshape(ns * a, ns * b).astype(jnp.bfloat16))

    w2f = conv2_w.astype(jnp.float32)
    w2full = jnp.zeros((mid, mid, K), jnp.float32)
    for g in range(groups):
        c0 = g * cin_g
        w2full = w2full.at[c0:c0 + cin_g, c0:c0 + cin_g, :].set(w2f[c0:c0 + cin_g])
    w2b = (jnp.einsum('ij,abk->iakjb', eye, w2full)
           .reshape(ns * mid, K * ns * mid).astype(jnp.bfloat16))

    colA = jnp.stack([s1, t1, s2, t2, s3, t3,
                      se_fc1_b.astype(jnp.float32)]).reshape(7, mid, 1)
    colB = jnp.stack([conv3_b.astype(jnp.float32),
                      se_fc2_b.astype(jnp.float32)]).reshape(2, Cout, 1)

    li = jnp.arange(_SEL_CHUNK)[:, None]
    qi = jnp.arange(_SEL_CHUNK // stride)[None, :]
    sel = jnp.stack([(li == stride * qi).astype(jnp.bfloat16),
                     (li == stride * qi + 1).astype(jnp.bfloat16)])

    xs = x.reshape(N // ns, ns * Cin, L)
    grid = (N // ns,)
    bs = pl.BlockSpec
    R1, R3 = ns * mid, ns * Cout
    ones_se = jnp.ones((L_out, 128), jnp.bfloat16)

    out = pl.pallas_call(
        _make_fused_kernel(ns, mid, Cout, Cin, L_out, K, L),
        out_shape=jax.ShapeDtypeStruct((N, Cout, L_out), jnp.float32),
        grid=grid,
        in_specs=[
            bs((1, ns * Cin, L), lambda n: (n, 0, 0)),
            bs((2, _SEL_CHUNK, _SEL_CHUNK // stride), lambda n: (0, 0, 0)),
            bs((7, mid, 1), lambda n: (0, 0, 0)),
            bs((2, Cout, 1), lambda n: (0, 0, 0)),
            bs((R1, ns * Cin), lambda n: (0, 0)),
            bs((R1, K * R1), lambda n: (0, 0)),
            bs((R3, R1), lambda n: (0, 0)),
            bs((ns * half, R3), lambda n: (0, 0)),
            bs((R3, ns * half), lambda n: (0, 0)),
            bs((L_out, 128), lambda n: (0, 0)),
        ],
        out_specs=bs((ns, Cout, L_out), lambda n: (n, 0, 0)),
        compiler_params=pltpu.CompilerParams(
            dimension_semantics=("parallel",)),
    )(xs, sel, colA, colB, kron(conv1_w[:, :, 0]), w2b,
      kron(conv3_w[:, :, 0]), kron(se_fc1_w), kron(se_fc2_w), ones_se)
    return out


# speedup vs baseline: 1.1030x; 1.0411x over previous
import jax
import jax.numpy as jnp
from jax.experimental import pallas as pl
from jax.experimental.pallas import tpu as pltpu

_SEL_CHUNK = 256


def _sigmoid(x):
    return 0.5 * jnp.tanh(0.5 * x) + 0.5


def _swish(x):
    return x * _sigmoid(x)


def _bn_affine(gamma, beta, mean, var, eps):
    s = gamma / jnp.sqrt(var + eps)
    return s, beta - mean * s


def _bdot(a, b):
    return jnp.dot(a, b, preferred_element_type=jnp.float32)


def _make_fused_kernel(ns, mid, cout, cin, l_out, K, L):
    inv_l = 1.0 / float(l_out)
    lc = (cout - cin) // 2
    rc = cout - cin - lc
    R1 = ns * mid
    R3 = ns * cout
    nch = L // _SEL_CHUNK

    def body(x_ref, sel_ref, colA_ref, colB_ref, w1_ref, w2_ref,
             w3_ref, wf1_ref, wf2_ref, ones_ref, o_ref):
        w1 = w1_ref[...]
        w2 = w2_ref[...]
        w3 = w3_ref[...]
        wf1 = wf1_ref[...]
        wf2 = wf2_ref[...]
        ones = ones_ref[...]
        cA = jnp.tile(colA_ref[...], (1, ns, 1))
        s1, t1, t2, t3, bf1 = [cA[i] for i in range(5)]
        cB = jnp.tile(colB_ref[...], (1, ns, 1))
        b3, bf2 = cB[0], cB[1]
        x16 = x_ref[0].astype(jnp.bfloat16)
        xc = [x16[:, t * _SEL_CHUNK:(t + 1) * _SEL_CHUNK] for t in range(nch)]
        xe = jnp.concatenate([_bdot(c, sel_ref[0]) for c in xc], axis=1)
        xo = jnp.concatenate([_bdot(c, sel_ref[1]) for c in xc], axis=1)
        h = []
        for xs in (xe, xo):
            a = _swish(s1 * xs + t1).astype(jnp.bfloat16)
            h.append(_swish(_bdot(w1, a) + t2).astype(jnp.bfloat16))
        he, ho = h
        z1c = jnp.zeros((R1, 1), jnp.bfloat16)
        taps = (
            jnp.concatenate([z1c, ho[:, :l_out - 1]], axis=1),
            he,
            ho,
            jnp.concatenate([he[:, 1:], z1c], axis=1),
            jnp.concatenate([ho[:, 1:], z1c], axis=1),
        )
        y2 = _bdot(w2[:, 0:R1], taps[0])
        for k in range(1, K):
            y2 = y2 + _bdot(w2[:, k * R1:(k + 1) * R1], taps[k])
        h3 = _swish(y2 + t3).astype(jnp.bfloat16)
        y3 = _bdot(w3, h3) + b3
        se_b = _bdot(y3.astype(jnp.bfloat16), ones) * inv_l
        zz = _swish(_bdot(wf1, se_b.astype(jnp.bfloat16)) + bf1)
        z2 = _bdot(wf2, zz.astype(jnp.bfloat16)) + bf2
        gate = _sigmoid(z2)[:, 0:1]
        ident = jnp.maximum(xe, xo)
        idp = jnp.pad(ident.reshape(ns, cin, l_out),
                      ((0, 0), (lc, rc), (0, 0))).reshape(R3, l_out)
        o_ref[...] = (y3 * gate + idp).reshape(ns, cout, l_out)
    return body


def kernel(x, bn1_g, bn1_b, bn1_m, bn1_v, conv1_w, conv1_b,
           bn2_g, bn2_b, bn2_m, bn2_v, conv2_w, conv2_b,
           bn3_g, bn3_b, bn3_m, bn3_v, conv3_w, conv3_b,
           se_fc1_w, se_fc1_b, se_fc2_w, se_fc2_b):
    K, stride, groups = 5, 2, 2
    bn_eps = 1e-5
    N, Cin, L = x.shape
    mid = conv1_w.shape[0]
    Cout = conv3_w.shape[0]
    half = se_fc1_w.shape[0]
    cin_g = mid // groups

    s1, t1 = _bn_affine(bn1_g, bn1_b, bn1_m, bn1_v, bn_eps)
    s2, t2 = _bn_affine(bn2_g, bn2_b, bn2_m, bn2_v, bn_eps)
    s3, t3 = _bn_affine(bn3_g, bn3_b, bn3_m, bn3_v, bn_eps)
    t2 = t2 + s2 * conv1_b
    t3 = t3 + s3 * conv2_b

    L_out = -(-L // stride)
    p = max(0, (L_out - 1) * stride + K - L)
    assert p // 2 == 1 and L % 2 == 0 and stride == 2 and L % _SEL_CHUNK == 0

    ns = next(c for c in (16, 8, 4, 2, 1) if N % c == 0)
    eye = jnp.eye(ns, dtype=jnp.float32)

    def kron(w):
        a, b = w.shape
        return (jnp.einsum('ij,ab->iajb', eye, w.astype(jnp.float32))
                .reshape(ns * a, ns * b).astype(jnp.bfloat16))

    w2f = conv2_w.astype(jnp.float32)
    w2full = jnp.zeros((mid, mid, K), jnp.float32)
    for g in range(groups):
        c0 = g * cin_g
        w2full = w2full.at[c0:c0 + cin_g, c0:c0 + cin_g, :].set(w2f[c0:c0 + cin_g])
    w2b = (jnp.einsum('ij,abk->iakjb', eye, s3[:, None, None] * w2full)
           .reshape(ns * mid, K * ns * mid).astype(jnp.bfloat16))

    colA = jnp.stack([s1, t1, t2, t3,
                      se_fc1_b.astype(jnp.float32)]).reshape(5, mid, 1)
    colB = jnp.stack([conv3_b.astype(jnp.float32),
                      se_fc2_b.astype(jnp.float32)]).reshape(2, Cout, 1)

    li = jnp.arange(_SEL_CHUNK)[:, None]
    qi = jnp.arange(_SEL_CHUNK // stride)[None, :]
    sel = jnp.stack([(li == stride * qi).astype(jnp.bfloat16),
                     (li == stride * qi + 1).astype(jnp.bfloat16)])

    xs = x.reshape(N // ns, ns * Cin, L)
    grid = (N // ns,)
    bs = pl.BlockSpec
    R1, R3 = ns * mid, ns * Cout
    ones_se = jnp.ones((L_out, 128), jnp.bfloat16)

    out = pl.pallas_call(
        _make_fused_kernel(ns, mid, Cout, Cin, L_out, K, L),
        out_shape=jax.ShapeDtypeStruct((N, Cout, L_out), jnp.float32),
        grid=grid,
        in_specs=[
            bs((1, ns * Cin, L), lambda n: (n, 0, 0)),
            bs((2, _SEL_CHUNK, _SEL_CHUNK // stride), lambda n: (0, 0, 0)),
            bs((5, mid, 1), lambda n: (0, 0, 0)),
            bs((2, Cout, 1), lambda n: (0, 0, 0)),
            bs((R1, ns * Cin), lambda n: (0, 0)),
            bs((R1, K * R1), lambda n: (0, 0)),
            bs((R3, R1), lambda n: (0, 0)),
            bs((ns * half, R3), lambda n: (0, 0)),
            bs((R3, ns * half), lambda n: (0, 0)),
            bs((L_out, 128), lambda n: (0, 0)),
        ],
        out_specs=bs((ns, Cout, L_out), lambda n: (n, 0, 0)),
        compiler_params=pltpu.CompilerParams(
            dimension_semantics=("parallel",)),
    )(xs, sel, colA, colB, kron(s2[:, None] * conv1_w[:, :, 0]), w2b,
      kron(conv3_w[:, :, 0]), kron(se_fc1_w), kron(se_fc2_w), ones_se)
    return out
